# baseline (speedup 1.0000x reference)
import jax
import jax.numpy as jnp
from jax.experimental import pallas as pl
from jax.experimental.pallas import tpu as pltpu

_VMEM_LIMIT = 60 * 1024 * 1024
_ENC_FRACS = (2, 4, 6, 4)
_DEC_FRACS = (4, 4, 4, 3, 1)


def _gae_kernel(x_ref, adj_ref, w0_ref, w1_ref, out_ref,
                xv_ref, abuf_ref, t_ref, adjb_ref, u_ref, z_ref, obuf_ref,
                xsem, asem, osem):
    g = pl.program_id(0)
    n = adjb_ref.shape[0]
    nenc = len(_ENC_FRACS)
    h2 = z_ref.shape[1]

    tsizes = [(n // 16) * f for f in _ENC_FRACS]
    toffs = [sum(tsizes[:i]) for i in range(len(tsizes))]

    @pl.when(g == 0)
    def _():
        pltpu.make_async_copy(x_ref, xv_ref, xsem).start()
        for i in range(len(tsizes)):
            pltpu.make_async_copy(
                adj_ref.at[pl.ds(toffs[i], tsizes[i]), :],
                abuf_ref.at[pl.ds(toffs[i], tsizes[i]), :],
                asem.at[i],
            ).start()

    for i in range(len(tsizes)):
        @pl.when(g == i)
        def _(i=i):
            off, tm_i = toffs[i], tsizes[i]

            if i == 0:
                pltpu.make_async_copy(x_ref, xv_ref, xsem).wait()
                x = xv_ref[...].astype(jnp.bfloat16)
                w0 = w0_ref[...].astype(jnp.bfloat16)
                t_ref[...] = jnp.dot(
                    x, w0, preferred_element_type=jnp.float32
                ).astype(jnp.bfloat16)

            pltpu.make_async_copy(
                abuf_ref.at[pl.ds(off, tm_i), :],
                abuf_ref.at[pl.ds(off, tm_i), :],
                asem.at[i],
            ).wait()
            adj_b = abuf_ref[pl.ds(off, tm_i), :].astype(jnp.bfloat16)
            adjb_ref[pl.ds(off, tm_i), :] = adj_b

            h = jnp.dot(adj_b, t_ref[...], preferred_element_type=jnp.float32)
            h = jnp.maximum(h, 0.0).astype(jnp.bfloat16)
            w1 = w1_ref[...].astype(jnp.bfloat16)
            u = jnp.dot(h, w1, preferred_element_type=jnp.float32)
            u_ref[pl.ds(off, tm_i), :] = jnp.pad(
                u, ((0, 0), (0, u_ref.shape[1] - h2))
            ).astype(jnp.bfloat16)

    @pl.when(g == nenc - 1)
    def _():
        z_wide = jnp.dot(
            adjb_ref[...], u_ref[...], preferred_element_type=jnp.float32
        )
        z_ref[...] = z_wide[:, :h2].astype(jnp.bfloat16)

    bsizes = [(n // 16) * f for f in _DEC_FRACS]
    boffs = [sum(bsizes[:i]) for i in range(len(bsizes))]
    for b in range(len(bsizes)):
        @pl.when(g == nenc + b)
        def _(b=b):
            off, bs = boffs[b], bsizes[b]
            slot = b % 2
            if b >= 2:
                pbs = bsizes[b - 2]
                pltpu.make_async_copy(
                    obuf_ref.at[slot, pl.ds(0, pbs), :],
                    obuf_ref.at[slot, pl.ds(0, pbs), :],
                    osem.at[slot],
                ).wait()
            zb = z_ref[pl.ds(off, bs), :]
            logits = jax.lax.dot_general(
                zb, z_ref[...],
                dimension_numbers=(((1,), (1,)), ((), ())),
                preferred_element_type=jnp.float32,
            )
            obuf_ref.at[slot, pl.ds(0, bs), :][...] = jax.nn.sigmoid(logits)
            pltpu.make_async_copy(
                obuf_ref.at[slot, pl.ds(0, bs), :],
                out_ref.at[pl.ds(off, bs), :],
                osem.at[slot],
            ).start()
            if b == len(bsizes) - 1:
                pbs = bsizes[b - 1]
                pltpu.make_async_copy(
                    obuf_ref.at[(b - 1) % 2, pl.ds(0, pbs), :],
                    obuf_ref.at[(b - 1) % 2, pl.ds(0, pbs), :],
                    osem.at[(b - 1) % 2],
                ).wait()
                pltpu.make_async_copy(
                    obuf_ref.at[slot, pl.ds(0, bs), :],
                    obuf_ref.at[slot, pl.ds(0, bs), :],
                    osem.at[slot],
                ).wait()


@jax.jit
def kernel(x, adj, w0, w1):
    n, in_dim = x.shape
    h1 = w0.shape[1]
    h2 = w1.shape[1]

    nenc = len(_ENC_FRACS)
    ndec = len(_DEC_FRACS)
    bmax = (n // 16) * max(_DEC_FRACS)
    h2w = max(h2, 256)

    a_pred = pl.pallas_call(
        _gae_kernel,
        out_shape=jax.ShapeDtypeStruct((n, n), jnp.float32),
        grid=(nenc + ndec,),
        in_specs=[
            pl.BlockSpec(memory_space=pltpu.MemorySpace.HBM),
            pl.BlockSpec(memory_space=pltpu.MemorySpace.HBM),
            pl.BlockSpec((in_dim, h1), lambda g: (0, 0)),
            pl.BlockSpec((h1, h2), lambda g: (0, 0)),
        ],
        out_specs=pl.BlockSpec(memory_space=pltpu.MemorySpace.HBM),
        scratch_shapes=[
            pltpu.VMEM((n, in_dim), jnp.float32),
            pltpu.VMEM((n, n), jnp.float32),
            pltpu.VMEM((n, h1), jnp.bfloat16),
            pltpu.VMEM((n, n), jnp.bfloat16),
            pltpu.VMEM((n, h2w), jnp.bfloat16),
            pltpu.VMEM((n, h2), jnp.bfloat16),
            pltpu.VMEM((2, bmax, n), jnp.float32),
            pltpu.SemaphoreType.DMA,
            pltpu.SemaphoreType.DMA((nenc,)),
            pltpu.SemaphoreType.DMA((2,)),
        ],
        compiler_params=pltpu.CompilerParams(
            dimension_semantics=("arbitrary",),
            vmem_limit_bytes=_VMEM_LIMIT,
        ),
    )(x, adj, w0, w1)

    return a_pred


# speedup vs baseline: 1.0866x; 1.0866x over previous
import jax
import jax.numpy as jnp
from jax.experimental import pallas as pl
from jax.experimental.pallas import tpu as pltpu

_VMEM_LIMIT = 60 * 1024 * 1024
_ENC_TILE = 1024
_DEC_FRACS = (4, 4, 4, 3, 1)


def _gae_kernel(x_ref, adj_ref, w0_ref, w1_ref, out_ref,
                xv_ref, abuf_ref, t_ref, adjb_ref, u_ref, z_ref, obuf_ref,
                xsem, asem, osem):
    g = pl.program_id(0)
    n = adjb_ref.shape[0]
    tm = abuf_ref.shape[1]
    nenc = n // tm
    h2 = z_ref.shape[1]

    @pl.when(g == 0)
    def _():
        pltpu.make_async_copy(x_ref, xv_ref, xsem).start()
        pltpu.make_async_copy(
            adj_ref.at[pl.ds(0, tm), :], abuf_ref.at[0], asem.at[0]
        ).start()

    @pl.when(g < nenc)
    def _():
        slot = jax.lax.rem(g, 2)
        nslot = jax.lax.rem(g + 1, 2)

        @pl.when(g + 1 < nenc)
        def _():
            pltpu.make_async_copy(
                adj_ref.at[pl.ds((g + 1) * tm, tm), :],
                abuf_ref.at[nslot], asem.at[nslot],
            ).start()

        pltpu.make_async_copy(
            abuf_ref.at[slot], abuf_ref.at[slot], asem.at[slot]
        ).wait()
        adj_b = abuf_ref.at[slot][...].astype(jnp.bfloat16)
        adjb_ref[pl.ds(g * tm, tm), :] = adj_b

        @pl.when(g == 0)
        def _():
            pltpu.make_async_copy(x_ref, xv_ref, xsem).wait()
            x = xv_ref[...].astype(jnp.bfloat16)
            w0 = w0_ref[...].astype(jnp.bfloat16)
            t_ref[...] = jnp.dot(
                x, w0, preferred_element_type=jnp.float32
            ).astype(jnp.bfloat16)

        h = jnp.dot(adj_b, t_ref[...], preferred_element_type=jnp.float32)
        h = jnp.maximum(h, 0.0).astype(jnp.bfloat16)
        w1 = w1_ref[...].astype(jnp.bfloat16)
        u = jnp.dot(h, w1, preferred_element_type=jnp.float32)
        u_ref[pl.ds(g * tm, tm), :] = jnp.pad(
            u, ((0, 0), (0, u_ref.shape[1] - h2))
        ).astype(jnp.bfloat16)

    @pl.when(g == nenc - 1)
    def _():
        z_wide = jnp.dot(
            adjb_ref[...], u_ref[...], preferred_element_type=jnp.float32
        )
        z_ref[...] = z_wide[:, :h2].astype(jnp.bfloat16)

    bsizes = [(n // 16) * f for f in _DEC_FRACS]
    boffs = [sum(bsizes[:i]) for i in range(len(bsizes))]
    for b in range(len(bsizes)):
        @pl.when(g == nenc + b)
        def _(b=b):
            off, bs = boffs[b], bsizes[b]
            slot = b % 2
            if b >= 2:
                pbs = bsizes[b - 2]
                pltpu.make_async_copy(
                    obuf_ref.at[slot, pl.ds(0, pbs), :],
                    obuf_ref.at[slot, pl.ds(0, pbs), :],
                    osem.at[slot],
                ).wait()
            zb = z_ref[pl.ds(off, bs), :]
            logits = jax.lax.dot_general(
                zb, z_ref[...],
                dimension_numbers=(((1,), (1,)), ((), ())),
                preferred_element_type=jnp.float32,
            )
            obuf_ref.at[slot, pl.ds(0, bs), :][...] = jax.nn.sigmoid(logits)
            pltpu.make_async_copy(
                obuf_ref.at[slot, pl.ds(0, bs), :],
                out_ref.at[pl.ds(off, bs), :],
                osem.at[slot],
            ).start()
            if b == len(bsizes) - 1:
                pbs = bsizes[b - 1]
                pltpu.make_async_copy(
                    obuf_ref.at[(b - 1) % 2, pl.ds(0, pbs), :],
                    obuf_ref.at[(b - 1) % 2, pl.ds(0, pbs), :],
                    osem.at[(b - 1) % 2],
                ).wait()
                pltpu.make_async_copy(
                    obuf_ref.at[slot, pl.ds(0, bs), :],
                    obuf_ref.at[slot, pl.ds(0, bs), :],
                    osem.at[slot],
                ).wait()


@jax.jit
def kernel(x, adj, w0, w1):
    n, in_dim = x.shape
    h1 = w0.shape[1]
    h2 = w1.shape[1]

    tm = _ENC_TILE if n % _ENC_TILE == 0 else n
    nenc = n // tm
    ndec = len(_DEC_FRACS)
    bmax = (n // 16) * max(_DEC_FRACS)
    h2w = max(h2, 256)

    a_pred = pl.pallas_call(
        _gae_kernel,
        out_shape=jax.ShapeDtypeStruct((n, n), jnp.float32),
        grid=(nenc + ndec,),
        in_specs=[
            pl.BlockSpec(memory_space=pltpu.MemorySpace.HBM),
            pl.BlockSpec(memory_space=pltpu.MemorySpace.HBM),
            pl.BlockSpec((in_dim, h1), lambda g: (0, 0)),
            pl.BlockSpec((h1, h2), lambda g: (0, 0)),
        ],
        out_specs=pl.BlockSpec(memory_space=pltpu.MemorySpace.HBM),
        scratch_shapes=[
            pltpu.VMEM((n, in_dim), jnp.float32),
            pltpu.VMEM((2, tm, n), jnp.float32),
            pltpu.VMEM((n, h1), jnp.bfloat16),
            pltpu.VMEM((n, n), jnp.bfloat16),
            pltpu.VMEM((n, h2w), jnp.bfloat16),
            pltpu.VMEM((n, h2), jnp.bfloat16),
            pltpu.VMEM((2, bmax, n), jnp.float32),
            pltpu.SemaphoreType.DMA,
            pltpu.SemaphoreType.DMA((2,)),
            pltpu.SemaphoreType.DMA((2,)),
        ],
        compiler_params=pltpu.CompilerParams(
            dimension_semantics=("arbitrary",),
            vmem_limit_bytes=_VMEM_LIMIT,
        ),
    )(x, adj, w0, w1)

    return a_pred


# speedup vs baseline: 1.1230x; 1.0334x over previous
import jax
import jax.numpy as jnp
from jax.experimental import pallas as pl
from jax.experimental.pallas import tpu as pltpu

_VMEM_LIMIT = 60 * 1024 * 1024
_ENC_TILE = 1024
_DEC_FRACS = (4, 4, 4, 3, 1)


def _gae_kernel(x_ref, adj_ref, w0_ref, w1_ref, out_ref,
                xv_ref, abuf_ref, t_ref, adjb_ref, u_ref, z_ref, obuf_ref,
                xsem, asem, osem):
    g = pl.program_id(0)
    n = adjb_ref.shape[0]
    tm = abuf_ref.shape[1]
    nenc = n // tm
    h2 = z_ref.shape[1]

    @pl.when(g == 0)
    def _():
        pltpu.make_async_copy(
            adj_ref.at[pl.ds(0, tm), :], abuf_ref.at[0], asem.at[0]
        ).start()
        pltpu.make_async_copy(x_ref, xv_ref, xsem).start()

    @pl.when(g < nenc)
    def _():
        slot = jax.lax.rem(g, 2)
        nslot = jax.lax.rem(g + 1, 2)

        @pl.when(g + 1 < nenc)
        def _():
            pltpu.make_async_copy(
                adj_ref.at[pl.ds((g + 1) * tm, tm), :],
                abuf_ref.at[nslot], asem.at[nslot],
            ).start()

        pltpu.make_async_copy(
            abuf_ref.at[slot], abuf_ref.at[slot], asem.at[slot]
        ).wait()
        adj_b = abuf_ref.at[slot][...].astype(jnp.bfloat16)
        adjb_ref[pl.ds(g * tm, tm), :] = adj_b

        @pl.when(g == 0)
        def _():
            pltpu.make_async_copy(x_ref, xv_ref, xsem).wait()
            x = xv_ref[...].astype(jnp.bfloat16)
            w0 = w0_ref[...].astype(jnp.bfloat16)
            t_ref[...] = jnp.dot(
                x, w0, preferred_element_type=jnp.float32
            ).astype(jnp.bfloat16)

        h = jnp.dot(adj_b, t_ref[...], preferred_element_type=jnp.float32)
        h = jnp.maximum(h, 0.0).astype(jnp.bfloat16)
        w1 = w1_ref[...].astype(jnp.bfloat16)
        u = jnp.dot(h, w1, preferred_element_type=jnp.float32)
        u_ref[pl.ds(g * tm, tm), :] = jnp.pad(
            u, ((0, 0), (0, u_ref.shape[1] - h2))
        ).astype(jnp.bfloat16)

    @pl.when(g == nenc - 1)
    def _():
        z_wide = jnp.dot(
            adjb_ref[...], u_ref[...], preferred_element_type=jnp.float32
        )
        z_ref[...] = z_wide[:, :h2].astype(jnp.bfloat16)

    bsizes = [(n // 16) * f for f in _DEC_FRACS]
    boffs = [sum(bsizes[:i]) for i in range(len(bsizes))]
    for b in range(len(bsizes)):
        @pl.when(g == nenc + b)
        def _(b=b):
            off, bs = boffs[b], bsizes[b]
            slot = b % 2
            if b >= 2:
                pbs = bsizes[b - 2]
                pltpu.make_async_copy(
                    obuf_ref.at[slot, pl.ds(0, pbs), :],
                    obuf_ref.at[slot, pl.ds(0, pbs), :],
                    osem.at[slot],
                ).wait()
            zb = z_ref[pl.ds(off, bs), :]
            logits = jax.lax.dot_general(
                zb, z_ref[...],
                dimension_numbers=(((1,), (1,)), ((), ())),
                preferred_element_type=jnp.float32,
            )
            obuf_ref.at[slot, pl.ds(0, bs), :][...] = jax.nn.sigmoid(logits)
            pltpu.make_async_copy(
                obuf_ref.at[slot, pl.ds(0, bs), :],
                out_ref.at[pl.ds(off, bs), :],
                osem.at[slot],
            ).start()
            if b == len(bsizes) - 1:
                pbs = bsizes[b - 1]
                pltpu.make_async_copy(
                    obuf_ref.at[(b - 1) % 2, pl.ds(0, pbs), :],
                    obuf_ref.at[(b - 1) % 2, pl.ds(0, pbs), :],
                    osem.at[(b - 1) % 2],
                ).wait()
                pltpu.make_async_copy(
                    obuf_ref.at[slot, pl.ds(0, bs), :],
                    obuf_ref.at[slot, pl.ds(0, bs), :],
                    osem.at[slot],
                ).wait()


@jax.jit
def kernel(x, adj, w0, w1):
    n, in_dim = x.shape
    h1 = w0.shape[1]
    h2 = w1.shape[1]

    tm = _ENC_TILE if n % _ENC_TILE == 0 else n
    nenc = n // tm
    ndec = len(_DEC_FRACS)
    bmax = (n // 16) * max(_DEC_FRACS)
    h2w = max(h2, 256)

    a_pred = pl.pallas_call(
        _gae_kernel,
        out_shape=jax.ShapeDtypeStruct((n, n), jnp.float32),
        grid=(nenc + ndec,),
        in_specs=[
            pl.BlockSpec(memory_space=pltpu.MemorySpace.HBM),
            pl.BlockSpec(memory_space=pltpu.MemorySpace.HBM),
            pl.BlockSpec((in_dim, h1), lambda g: (0, 0)),
            pl.BlockSpec((h1, h2), lambda g: (0, 0)),
        ],
        out_specs=pl.BlockSpec(memory_space=pltpu.MemorySpace.HBM),
        scratch_shapes=[
            pltpu.VMEM((n, in_dim), jnp.float32),
            pltpu.VMEM((2, tm, n), jnp.float32),
            pltpu.VMEM((n, h1), jnp.bfloat16),
            pltpu.VMEM((n, n), jnp.bfloat16),
            pltpu.VMEM((n, h2w), jnp.bfloat16),
            pltpu.VMEM((n, h2), jnp.bfloat16),
            pltpu.VMEM((2, bmax, n), jnp.float32),
            pltpu.SemaphoreType.DMA,
            pltpu.SemaphoreType.DMA((2,)),
            pltpu.SemaphoreType.DMA((2,)),
        ],
        compiler_params=pltpu.CompilerParams(
            dimension_semantics=("arbitrary",),
            vmem_limit_bytes=_VMEM_LIMIT,
        ),
    )(x, adj, w0, w1)

    return a_pred
